# baseline (speedup 1.0000x reference)
import functools

import jax
import jax.numpy as jnp
from jax import lax
from jax.experimental import pallas as pl
from jax.experimental.pallas import tpu as pltpu
from jax.experimental.pallas import tpu_sc as plsc

VOCAB = 100000
EMB = 64
SEQ = 200
BATCH = 1024
B = BATCH * SEQ
HALF = BATCH // 2

NC, NS, L = 2, 16, 16
NW = NC * NS
CHUNK = HALF
NSLAB = SEQ * 2
SLABS_PER_W = -(-NSLAB // NW)

_DK = 2048
_NBLK = (VOCAB + _DK - 1) // _DK
VOCAB_PAD = _NBLK * _DK


def _comb_body(seg_ref, pos_ref, out_ref):
    seg = seg_ref[...]
    pos = pos_ref[...]
    out_ref[...] = (seg[:, None, :] + pos[None, :, :]).reshape(3 * SEQ, EMB)


def _make_comb(seg_table, pos_emb):
    return pl.pallas_call(
        _comb_body,
        out_shape=jax.ShapeDtypeStruct((3 * SEQ, EMB), jnp.float32),
    )(seg_table, pos_emb)


def _detile_body(in_ref, out_ref):
    w = in_ref[...].T
    z = w.reshape(2, _DK // 2, EMB)
    out_ref[:, 0:EMB] = z[0]
    out_ref[:, EMB:2 * EMB] = z[1]


def _detile_table(tok_t):
    return pl.pallas_call(
        _detile_body,
        grid=(_NBLK,),
        in_specs=[pl.BlockSpec((EMB, _DK), lambda i: (0, i))],
        out_specs=pl.BlockSpec((_DK // 2, 2 * EMB), lambda i: (i, 0)),
        out_shape=jax.ShapeDtypeStruct((VOCAB_PAD // 2, 2 * EMB), jnp.float32),
    )(tok_t)


_TP = 2


def _transpose_body(in_ref, out_ref):
    for i in range(_TP):
        y = in_ref[pl.ds(i * HALF, HALF), :].T
        z = y.reshape(2, EMB, HALF)
        out_ref[i, :, 0:HALF] = z[0]
        out_ref[i, :, HALF:BATCH] = z[1]


def _transpose_out(pairs):
    return pl.pallas_call(
        _transpose_body,
        grid=(SEQ // _TP,),
        in_specs=[pl.BlockSpec((_TP * HALF, 2 * EMB), lambda i: (i, 0))],
        out_specs=pl.BlockSpec((_TP, EMB, BATCH), lambda i: (i, 0, 0)),
        out_shape=jax.ShapeDtypeStruct((SEQ, EMB, BATCH), jnp.float32),
    )(pairs)


_mesh = plsc.VectorSubcoreMesh(core_axis_name="c", subcore_axis_name="s")


@functools.partial(
    pl.kernel,
    mesh=_mesh,
    compiler_params=pltpu.CompilerParams(use_tc_tiling_on_sc=False),
    out_type=jax.ShapeDtypeStruct((B // 2, 2, EMB), jnp.float32),
    scratch_types=[
        pltpu.VMEM((CHUNK,), jnp.int32),
        pltpu.VMEM((CHUNK,), jnp.int32),
        pltpu.VMEM((CHUNK,), jnp.int32),
        pltpu.VMEM((CHUNK,), jnp.int32),
        pltpu.VMEM((CHUNK, EMB), jnp.float32),
        pltpu.VMEM((CHUNK, EMB), jnp.float32),
        pltpu.SemaphoreType.DMA,
        pltpu.SemaphoreType.DMA,
        pltpu.SemaphoreType.DMA,
        pltpu.SemaphoreType.DMA,
        pltpu.SemaphoreType.DMA,
        pltpu.SemaphoreType.DMA,
    ],
)
def _sc_embed(inpT_hbm, segT_hbm, tok_hbm, comb_hbm, out_hbm,
              idx0, idx1, cidx0, cidx1, rows0, rows1,
              semt0, semt1, sema0, sema1, semw0, semw1):
    idx = (idx0, idx1)
    cidx = (cidx0, cidx1)
    rows = (rows0, rows1)
    semt = (semt0, semt1)
    sema = (sema0, sema1)
    semw = (semw0, semw1)

    wid = lax.axis_index("s") * NC + lax.axis_index("c")

    def slab_id(k):
        s = wid + NW * k
        if (k + 1) * NW > NSLAB:
            s = jnp.where(s < NSLAB, s, s - NW)
        return s

    def stage(k, p):
        s = slab_id(k)
        off = s * CHUNK
        pltpu.sync_copy(inpT_hbm.at[pl.ds(off, CHUNK)], idx[p])
        pltpu.sync_copy(segT_hbm.at[pl.ds(off, CHUNK)], cidx[p])
        pos = s >> 1

        def remap_body(i, _):
            sl = pl.ds(i * L, L)
            t = idx[p][sl]
            idx[p][sl] = (((t >> 11) << 11) + ((t & 1023) << 1)
                          + ((t >> 10) & 1))
            cidx[p][sl] = cidx[p][sl] * SEQ + pos
            return 0

        lax.fori_loop(0, CHUNK // L, remap_body, 0)
        return pltpu.async_copy(tok_hbm.at[idx[p]], rows[p], semt[p])

    def issue_wb(k, p):
        s = slab_id(k)
        return pltpu.async_copy(
            rows[p], out_hbm.at[pl.ds((s >> 1) * HALF, HALF), s & 1], semw[p])

    tok_cp = [None, None]
    wb_cp = [None, None]
    tok_cp[0] = stage(0, 0)
    for k in range(SLABS_PER_W):
        p = k % 2
        q = p ^ 1
        tok_cp[p].wait()
        add_cp = pltpu.async_copy(comb_hbm.at[cidx[p]], rows[p], sema[p], add=True)
        if k + 1 < SLABS_PER_W:
            if wb_cp[q] is not None:
                wb_cp[q].wait()
            tok_cp[q] = stage(k + 1, q)
        add_cp.wait()
        wb_cp[p] = issue_wb(k, p)
    wb_cp[(SLABS_PER_W - 1) % 2].wait()
    wb_cp[SLABS_PER_W % 2].wait()


def kernel(input, segment_label, tok_table, seg_table, pos_emb):
    comb = _make_comb(seg_table, pos_emb)
    tok_lin = _detile_table(tok_table.T)
    out_rows = _sc_embed(input.T.reshape(-1), segment_label.T.reshape(-1),
                         tok_lin.reshape(VOCAB_PAD, EMB), comb)
    out_peb = _transpose_out(out_rows.reshape(SEQ * HALF, 2 * EMB))
    return jnp.transpose(out_peb, (2, 0, 1))


# speedup vs baseline: 2.5106x; 2.5106x over previous
import functools

import jax
import jax.numpy as jnp
from jax import lax
from jax.experimental import pallas as pl
from jax.experimental.pallas import tpu as pltpu
from jax.experimental.pallas import tpu_sc as plsc

VOCAB = 100000
EMB = 64
SEQ = 200
BATCH = 1024
B = BATCH * SEQ

NC, NS, L = 2, 16, 16
NW = NC * NS
ROWS_PER_W = B // NW
CHUNK = 800
NCHUNK = ROWS_PER_W // CHUNK


def _comb_body(seg_ref, pos_ref, out_ref):
    seg = seg_ref[...]
    pos = pos_ref[...]
    out_ref[...] = (seg[:, None, :] + pos[None, :, :]).reshape(3 * SEQ, EMB)


def _make_comb(seg_table, pos_emb):
    return pl.pallas_call(
        _comb_body,
        out_shape=jax.ShapeDtypeStruct((3 * SEQ, EMB), jnp.float32),
    )(seg_table, pos_emb)


_mesh = plsc.VectorSubcoreMesh(core_axis_name="c", subcore_axis_name="s")


@functools.partial(
    pl.kernel,
    mesh=_mesh,
    compiler_params=pltpu.CompilerParams(use_tc_tiling_on_sc=False),
    out_type=jax.ShapeDtypeStruct((B, EMB), jnp.float32),
    scratch_types=[
        pltpu.VMEM((CHUNK,), jnp.int32),
        pltpu.VMEM((CHUNK,), jnp.int32),
        pltpu.VMEM((CHUNK,), jnp.int32),
        pltpu.VMEM((CHUNK,), jnp.int32),
        pltpu.VMEM((CHUNK,), jnp.int32),
        pltpu.VMEM((CHUNK,), jnp.int32),
        pltpu.VMEM((CHUNK, EMB), jnp.float32),
        pltpu.VMEM((CHUNK, EMB), jnp.float32),
        pltpu.SemaphoreType.DMA,
        pltpu.SemaphoreType.DMA,
        pltpu.SemaphoreType.DMA,
        pltpu.SemaphoreType.DMA,
        pltpu.SemaphoreType.DMA,
        pltpu.SemaphoreType.DMA,
    ],
)
def _sc_embed(inp_hbm, seg_hbm, tok_hbm, comb_hbm, out_hbm,
              idx0, idx1, seg0, seg1, cidx0, cidx1, tb0, tb1,
              semt0, semt1, sema0, sema1, semw0, semw1):
    idx = (idx0, idx1)
    seg = (seg0, seg1)
    cidx = (cidx0, cidx1)
    tb = (tb0, tb1)
    semt = (semt0, semt1)
    sema = (sema0, sema1)
    semw = (semw0, semw1)

    wid = lax.axis_index("s") * NC + lax.axis_index("c")
    base = wid * ROWS_PER_W

    def stage(c, p):
        off = base + c * CHUNK
        pltpu.sync_copy(inp_hbm.at[pl.ds(off, CHUNK)], idx[p])
        pltpu.sync_copy(seg_hbm.at[pl.ds(off, CHUNK)], seg[p])

        def cidx_body(i, _):
            s = seg[p][pl.ds(i * L, L)]
            pos = (i * L + lax.iota(jnp.int32, L)) % SEQ
            cidx[p][pl.ds(i * L, L)] = s * SEQ + pos
            return 0

        lax.fori_loop(0, CHUNK // L, cidx_body, 0)
        return pltpu.async_copy(tok_hbm.at[idx[p]], tb[p], semt[p])

    tok_cp = [None, None]
    wb_cp = [None, None]
    tok_cp[0] = stage(0, 0)
    for c in range(NCHUNK):
        p = c % 2
        q = p ^ 1
        tok_cp[p].wait()
        add_cp = pltpu.async_copy(comb_hbm.at[cidx[p]], tb[p], sema[p], add=True)
        if c + 1 < NCHUNK:
            if wb_cp[q] is not None:
                wb_cp[q].wait()
            tok_cp[q] = stage(c + 1, q)
        add_cp.wait()
        wb_cp[p] = pltpu.async_copy(tb[p], out_hbm.at[pl.ds(base + c * CHUNK, CHUNK)],
                                    semw[p])
    wb_cp[0].wait()
    wb_cp[1].wait()


def kernel(input, segment_label, tok_table, seg_table, pos_emb):
    comb = _make_comb(seg_table, pos_emb)
    out = _sc_embed(input.reshape(-1), segment_label.reshape(-1),
                    tok_table, comb)
    return out.reshape(BATCH, SEQ, EMB)
